# baseline (speedup 1.0000x reference)
import functools

import jax
import jax.numpy as jnp
from jax import lax
from jax.experimental import pallas as pl
from jax.experimental.pallas import tpu as pltpu
from jax.experimental.pallas import tpu_sc as plsc

B = 16384
D = 128
L = 16
NVREG = D // L
CHUNK = 128

NC, NS = 2, 16
NW = NC * NS
BPW = B // NW
NCHUNK = BPW // CHUNK
IDX_ROWS = B // CHUNK
ROWS_PW = IDX_ROWS // NW

def _sc_partial_body(uidx_hbm, iidx_hbm, uemb_hbm, iemb_hbm, out_hbm,
                uidx_v, iidx_v, urows_v, irows_v, acc_v, sem0, sem1):
    wid = lax.axis_index("s") * NC + lax.axis_index("c")
    row0 = wid * ROWS_PW
    pltpu.sync_copy(uidx_hbm.at[pl.ds(row0, ROWS_PW)], uidx_v)
    pltpu.sync_copy(iidx_hbm.at[pl.ds(row0, ROWS_PW)], iidx_v)

    sems = (sem0, sem1)

    def fire(k):
        slot = k % 2
        hu = pltpu.async_copy(uemb_hbm.at[uidx_v.at[k]], urows_v.at[slot],
                              sems[slot])
        hi = pltpu.async_copy(iemb_hbm.at[iidx_v.at[k]], irows_v.at[slot],
                              sems[slot])
        return hu, hi

    accs = tuple(jnp.zeros((L,), jnp.float32) for _ in range(NVREG))
    pend = fire(0)
    for k in range(NCHUNK):
        nxt = fire(k + 1) if k + 1 < NCHUNK else None
        pend[0].wait()
        pend[1].wait()
        u_ref = urows_v.at[k % 2]
        i_ref = irows_v.at[k % 2]

        def body(b2, acc):
            b = b2 * 2
            acc = tuple(
                acc[j] + u_ref[b, pl.ds(j * L, L)] * i_ref[b, pl.ds(j * L, L)]
                for j in range(NVREG)
            )
            return tuple(
                acc[j]
                + u_ref[b + 1, pl.ds(j * L, L)] * i_ref[b + 1, pl.ds(j * L, L)]
                for j in range(NVREG)
            )

        accs = lax.fori_loop(0, CHUNK // 2, body, accs)
        pend = nxt

    for j in range(NVREG):
        acc_v[pl.ds(j * L, L)] = accs[j]
    pltpu.sync_copy(acc_v, out_hbm.at[wid])


@functools.cache
def _sc_partial():
    mesh = plsc.VectorSubcoreMesh(
        core_axis_name="c", subcore_axis_name="s",
        num_cores=NC, num_subcores=NS,
    )
    return pl.kernel(
        _sc_partial_body,
        mesh=mesh,
        out_type=jax.ShapeDtypeStruct((NW, D), jnp.float32),
        scratch_types=[
            pltpu.VMEM((ROWS_PW, CHUNK), jnp.int32),
            pltpu.VMEM((ROWS_PW, CHUNK), jnp.int32),
            pltpu.VMEM((2, CHUNK, D), jnp.float32),
            pltpu.VMEM((2, CHUNK, D), jnp.float32),
            pltpu.VMEM((D,), jnp.float32),
            pltpu.SemaphoreType.DMA,
            pltpu.SemaphoreType.DMA,
        ],
    )


def _finale(p_ref, v_ref, o_ref):
    s = jnp.sum(p_ref[...], axis=0, keepdims=True)
    pred = 1.0 / (1.0 + jnp.exp(-s))
    v = (v_ref[0, 0] - 1.0) * 0.25
    r = pred - v
    o_ref[...] = r * r


def kernel(users_index, items_index, value, user_embed, item_embed):
    uidx = users_index.astype(jnp.int32).reshape(IDX_ROWS, CHUNK)
    iidx = items_index.astype(jnp.int32).reshape(IDX_ROWS, CHUNK)
    partials = _sc_partial()(uidx, iidx, user_embed, item_embed)
    out = pl.pallas_call(
        _finale,
        out_shape=jax.ShapeDtypeStruct((1, D), jnp.float32),
    )(partials, value.reshape(1, 1).astype(jnp.float32))
    return out.reshape(D)


# speedup vs baseline: 1.0359x; 1.0359x over previous
import functools

import jax
import jax.numpy as jnp
from jax import lax
from jax.experimental import pallas as pl
from jax.experimental.pallas import tpu as pltpu
from jax.experimental.pallas import tpu_sc as plsc

B = 16384
D = 128
L = 16
NVREG = D // L
CHUNK = 64
NSLOT = 4

NC, NS = 2, 16
NW = NC * NS
BPW = B // NW
NCHUNK = BPW // CHUNK
IDX_ROWS = B // CHUNK
ROWS_PW = IDX_ROWS // NW


def _sc_partial_body(idx_hbm, uemb_hbm, iemb_hbm, out_hbm,
                     idx_v, urows_v, irows_v, acc_v, isem,
                     sem0, sem1, sem2, sem3):
    sems = (sem0, sem1, sem2, sem3)
    wid = lax.axis_index("s") * NC + lax.axis_index("c")
    row0 = wid * ROWS_PW
    pltpu.async_copy(idx_hbm.at[:, pl.ds(row0, ROWS_PW)], idx_v, isem).wait()

    def fire(k):
        slot = k % NSLOT
        hu = pltpu.async_copy(uemb_hbm.at[idx_v.at[0, k]], urows_v.at[slot],
                              sems[slot])
        hi = pltpu.async_copy(iemb_hbm.at[idx_v.at[1, k]], irows_v.at[slot],
                              sems[slot])
        return hu, hi

    accs = tuple(jnp.zeros((L,), jnp.float32) for _ in range(NVREG))
    pend = [fire(k) for k in range(NSLOT - 1)]
    for k in range(NCHUNK):
        if k + NSLOT - 1 < NCHUNK:
            pend.append(fire(k + NSLOT - 1))
        hu, hi = pend.pop(0)
        hu.wait()
        hi.wait()
        u_ref = urows_v.at[k % NSLOT]
        i_ref = irows_v.at[k % NSLOT]

        def body(b2, acc):
            b = b2 * 2
            acc = tuple(
                acc[j] + u_ref[b, pl.ds(j * L, L)] * i_ref[b, pl.ds(j * L, L)]
                for j in range(NVREG)
            )
            return tuple(
                acc[j]
                + u_ref[b + 1, pl.ds(j * L, L)] * i_ref[b + 1, pl.ds(j * L, L)]
                for j in range(NVREG)
            )

        accs = lax.fori_loop(0, CHUNK // 2, body, accs)

    for j in range(NVREG):
        acc_v[pl.ds(j * L, L)] = accs[j]
    pltpu.sync_copy(acc_v, out_hbm.at[wid])


@functools.cache
def _sc_partial():
    mesh = plsc.VectorSubcoreMesh(
        core_axis_name="c", subcore_axis_name="s",
        num_cores=NC, num_subcores=NS,
    )
    return pl.kernel(
        _sc_partial_body,
        mesh=mesh,
        out_type=jax.ShapeDtypeStruct((NW, D), jnp.float32),
        scratch_types=[
            pltpu.VMEM((2, ROWS_PW, CHUNK), jnp.int32),
            pltpu.VMEM((NSLOT, CHUNK, D), jnp.float32),
            pltpu.VMEM((NSLOT, CHUNK, D), jnp.float32),
            pltpu.VMEM((D,), jnp.float32),
            pltpu.SemaphoreType.DMA,
            pltpu.SemaphoreType.DMA,
            pltpu.SemaphoreType.DMA,
            pltpu.SemaphoreType.DMA,
            pltpu.SemaphoreType.DMA,
        ],
    )


def _finale(p_ref, v_ref, o_ref):
    s = jnp.sum(p_ref[...], axis=0, keepdims=True)
    pred = 1.0 / (1.0 + jnp.exp(-s))
    v = (v_ref[0, 0] - 1.0) * 0.25
    r = pred - v
    o_ref[...] = r * r


def kernel(users_index, items_index, value, user_embed, item_embed):
    idx = jnp.stack(
        [users_index.astype(jnp.int32), items_index.astype(jnp.int32)]
    ).reshape(2, IDX_ROWS, CHUNK)
    partials = _sc_partial()(idx, user_embed, item_embed)
    out = pl.pallas_call(
        _finale,
        out_shape=jax.ShapeDtypeStruct((1, D), jnp.float32),
    )(partials, value.reshape(1, 1).astype(jnp.float32))
    return out.reshape(D)


# speedup vs baseline: 1.0389x; 1.0029x over previous
import functools

import jax
import jax.numpy as jnp
from jax import lax
from jax.experimental import pallas as pl
from jax.experimental.pallas import tpu as pltpu
from jax.experimental.pallas import tpu_sc as plsc

B = 16384
D = 128
L = 16
NVREG = D // L
CHUNK = 64
NSLOT = 4

NC, NS = 2, 16
NW = NC * NS
BPW = B // NW
NCHUNK = BPW // CHUNK
IDX_ROWS = B // CHUNK
ROWS_PW = IDX_ROWS // NW


def _sc_partial_body(uidx_hbm, iidx_hbm, uemb_hbm, iemb_hbm, out_hbm,
                     uidx_v, iidx_v, urows_v, irows_v, acc_v, isem,
                     sem0, sem1, sem2, sem3):
    sems = (sem0, sem1, sem2, sem3)
    wid = lax.axis_index("s") * NC + lax.axis_index("c")
    row0 = wid * ROWS_PW
    hu_idx = pltpu.async_copy(uidx_hbm.at[pl.ds(row0, ROWS_PW)], uidx_v, isem)
    hi_idx = pltpu.async_copy(iidx_hbm.at[pl.ds(row0, ROWS_PW)], iidx_v, isem)
    hu_idx.wait()
    hi_idx.wait()

    def fire(k):
        slot = k % NSLOT
        hu = pltpu.async_copy(uemb_hbm.at[uidx_v.at[k]], urows_v.at[slot],
                              sems[slot])
        hi = pltpu.async_copy(iemb_hbm.at[iidx_v.at[k]], irows_v.at[slot],
                              sems[slot])
        return hu, hi

    accs = tuple(jnp.zeros((L,), jnp.float32) for _ in range(NVREG))
    pend = [fire(k) for k in range(NSLOT - 1)]
    for k in range(NCHUNK):
        if k + NSLOT - 1 < NCHUNK:
            pend.append(fire(k + NSLOT - 1))
        hu, hi = pend.pop(0)
        hu.wait()
        hi.wait()
        u_ref = urows_v.at[k % NSLOT]
        i_ref = irows_v.at[k % NSLOT]

        def body(b2, acc):
            b = b2 * 2
            acc = tuple(
                acc[j] + u_ref[b, pl.ds(j * L, L)] * i_ref[b, pl.ds(j * L, L)]
                for j in range(NVREG)
            )
            return tuple(
                acc[j]
                + u_ref[b + 1, pl.ds(j * L, L)] * i_ref[b + 1, pl.ds(j * L, L)]
                for j in range(NVREG)
            )

        accs = lax.fori_loop(0, CHUNK // 2, body, accs)

    for j in range(NVREG):
        acc_v[pl.ds(j * L, L)] = accs[j]
    pltpu.sync_copy(acc_v, out_hbm.at[wid])


@functools.cache
def _sc_partial():
    mesh = plsc.VectorSubcoreMesh(
        core_axis_name="c", subcore_axis_name="s",
        num_cores=NC, num_subcores=NS,
    )
    return pl.kernel(
        _sc_partial_body,
        mesh=mesh,
        out_type=jax.ShapeDtypeStruct((NW, D), jnp.float32),
        scratch_types=[
            pltpu.VMEM((ROWS_PW, CHUNK), jnp.int32),
            pltpu.VMEM((ROWS_PW, CHUNK), jnp.int32),
            pltpu.VMEM((NSLOT, CHUNK, D), jnp.float32),
            pltpu.VMEM((NSLOT, CHUNK, D), jnp.float32),
            pltpu.VMEM((D,), jnp.float32),
            pltpu.SemaphoreType.DMA,
            pltpu.SemaphoreType.DMA,
            pltpu.SemaphoreType.DMA,
            pltpu.SemaphoreType.DMA,
            pltpu.SemaphoreType.DMA,
        ],
    )


def _finale(p_ref, v_ref, o_ref):
    s = jnp.sum(p_ref[...], axis=0, keepdims=True)
    pred = 1.0 / (1.0 + jnp.exp(-s))
    v = (v_ref[0, 0] - 1.0) * 0.25
    r = pred - v
    o_ref[...] = r * r


def kernel(users_index, items_index, value, user_embed, item_embed):
    uidx = users_index.astype(jnp.int32).reshape(IDX_ROWS, CHUNK)
    iidx = items_index.astype(jnp.int32).reshape(IDX_ROWS, CHUNK)
    partials = _sc_partial()(uidx, iidx, user_embed, item_embed)
    out = pl.pallas_call(
        _finale,
        out_shape=jax.ShapeDtypeStruct((1, D), jnp.float32),
    )(partials, value.reshape(1, 1).astype(jnp.float32))
    return out.reshape(D)
